# Initial kernel scaffold; baseline (speedup 1.0000x reference)
import jax
import jax.numpy as jnp
from jax.experimental import pallas as pl


def kernel(x, edge_index, W1, b1, W2, b2, gru_wih_l0, gru_whh_l0, gru_bih_l0, gru_bhh_l0, gru_wih_l0_r, gru_whh_l0_r, gru_bih_l0_r, gru_bhh_l0_r, gru_wih_l1, gru_whh_l1, gru_bih_l1, gru_bhh_l1, gru_wih_l1_r, gru_whh_l1_r, gru_bih_l1_r, gru_bhh_l1_r, Wlin, blin):
    raise NotImplementedError("write your pallas kernel here")


# speedup vs baseline: 5.2553x; 5.2553x over previous
import functools

import jax
import jax.numpy as jnp
from jax import lax
from jax.experimental import pallas as pl
from jax.experimental.pallas import tpu as pltpu
from jax.experimental.pallas import tpu_sc as plsc

N = 10000
E = 160000
H = 64

NC = 2
NS = 16
NW = NC * NS
CH = 128
EPW = 5120
NCHUNK = EPW // CH
EPAD = NW * EPW
NACC = 10112
ROWS = NACC // NS

_mesh = plsc.VectorSubcoreMesh(core_axis_name="c", subcore_axis_name="s")


def _make_deg_kernel():
    @functools.partial(
        pl.kernel,
        mesh=_mesh,
        compiler_params=pltpu.CompilerParams(use_tc_tiling_on_sc=False),
        out_type=jax.ShapeDtypeStruct((NC, NACC, 16), jnp.float32),
        scratch_types=[
            pltpu.VMEM((NCHUNK, CH), jnp.int32),
            pltpu.VMEM((CH, 16), jnp.float32),
            pltpu.VMEM((ROWS, 16), jnp.float32),
            pltpu.VMEM_SHARED((NACC, 16), jnp.float32),
        ],
    )
    def deg_kernel(dst2d, ones_hbm, zeros_hbm, out, idx_d, ones_v, stripe_v, acc_sh):
        cid = lax.axis_index("c")
        sid = lax.axis_index("s")
        wid = cid * NS + sid
        pltpu.sync_copy(zeros_hbm.at[pl.ds(sid * ROWS, ROWS)], stripe_v)
        pltpu.sync_copy(stripe_v, acc_sh.at[pl.ds(sid * ROWS, ROWS)])
        pltpu.sync_copy(ones_hbm, ones_v)
        pltpu.sync_copy(dst2d.at[pl.ds(wid * NCHUNK, NCHUNK)], idx_d)
        plsc.subcore_barrier()

        def body(k, carry):
            pltpu.sync_copy(ones_v, acc_sh.at[idx_d.at[k]], add=True)
            return carry

        lax.fori_loop(0, NCHUNK, body, 0)
        plsc.subcore_barrier()
        pltpu.sync_copy(acc_sh.at[pl.ds(sid * ROWS, ROWS)], stripe_v)
        pltpu.sync_copy(stripe_v, out.at[cid, pl.ds(sid * ROWS, ROWS)])

    return deg_kernel


def _make_conv_kernel(F):
    @functools.partial(
        pl.kernel,
        mesh=_mesh,
        compiler_params=pltpu.CompilerParams(use_tc_tiling_on_sc=False),
        out_type=jax.ShapeDtypeStruct((NC, NACC, F), jnp.float32),
        scratch_types=[
            pltpu.VMEM((NCHUNK, CH), jnp.int32),
            pltpu.VMEM((NCHUNK, CH), jnp.int32),
            pltpu.VMEM((CH, F), jnp.float32),
            pltpu.VMEM((ROWS, F), jnp.float32),
            pltpu.VMEM_SHARED((NACC, F), jnp.float32),
            pltpu.SemaphoreType.DMA,
        ],
    )
    def conv_kernel(y_hbm, src2d, dst2d, zeros_hbm, out,
                    idx_s, idx_d, rows_v, stripe_v, acc_sh, sem):
        cid = lax.axis_index("c")
        sid = lax.axis_index("s")
        wid = cid * NS + sid
        pltpu.sync_copy(zeros_hbm.at[pl.ds(sid * ROWS, ROWS)], stripe_v)
        pltpu.sync_copy(stripe_v, acc_sh.at[pl.ds(sid * ROWS, ROWS)])
        pltpu.sync_copy(src2d.at[pl.ds(wid * NCHUNK, NCHUNK)], idx_s)
        pltpu.sync_copy(dst2d.at[pl.ds(wid * NCHUNK, NCHUNK)], idx_d)
        plsc.subcore_barrier()

        def body(k, carry):
            pltpu.async_copy(y_hbm.at[idx_s.at[k]], rows_v, sem).wait()
            pltpu.sync_copy(rows_v, acc_sh.at[idx_d.at[k]], add=True)
            return carry

        lax.fori_loop(0, NCHUNK, body, 0)
        plsc.subcore_barrier()
        pltpu.sync_copy(acc_sh.at[pl.ds(sid * ROWS, ROWS)], stripe_v)
        pltpu.sync_copy(stripe_v, out.at[cid, pl.ds(sid * ROWS, ROWS)])

    return conv_kernel


_deg_kernel = _make_deg_kernel()
_conv16 = _make_conv_kernel(16)
_conv32 = _make_conv_kernel(32)


def _tc1_body(x_ref, w1_ref, degp_ref, y1_ref, dinv_ref):
    deg = degp_ref[:, 0:1] + degp_ref[:, 1:2] + 1.0
    dinv = lax.rsqrt(deg)
    xw = jnp.dot(x_ref[...], w1_ref[...], preferred_element_type=jnp.float32)
    y1_ref[...] = dinv * xw
    dinv_ref[...] = dinv


def _tc2_body(acca_ref, accb_ref, y_ref, dinv_ref, b_ref, w2_ref, y2_ref):
    dinv = dinv_ref[...]
    h = acca_ref[...] + accb_ref[...] + y_ref[...]
    h = jax.nn.relu(dinv * h + b_ref[...])
    y2_ref[...] = dinv * jnp.dot(h, w2_ref[...], preferred_element_type=jnp.float32)


def _tc3_body(acca_ref, accb_ref, y_ref, dinv_ref, b_ref, h_ref):
    h = acca_ref[...] + accb_ref[...] + y_ref[...]
    h_ref[...] = jax.nn.relu(dinv_ref[...] * h + b_ref[...])


def _gi_body(haug_ref, wbig_ref, bcat_ref, gi_ref):
    gi_ref[...] = jnp.dot(haug_ref[...], wbig_ref[...],
                          preferred_element_type=jnp.float32) + bcat_ref[...]


def _scan_body(gi_ref, wblk_ref, bhh_ref, hall_ref):
    wblk = wblk_ref[...]
    bhh = bhh_ref[...]

    def step(t, h):
        gi_t = gi_ref[pl.ds(t, 1), :]
        gh = jnp.dot(h, wblk, preferred_element_type=jnp.float32) + bhh
        r = jax.nn.sigmoid(gi_t[:, 0:128] + gh[:, 0:128])
        z = jax.nn.sigmoid(gi_t[:, 128:256] + gh[:, 128:256])
        cand = jnp.tanh(gi_t[:, 256:384] + r * gh[:, 256:384])
        hn = (1.0 - z) * cand + z * h
        hall_ref[pl.ds(t, 1), :] = hn
        return hn

    lax.fori_loop(0, N, step, jnp.zeros((1, 2 * H), jnp.float32))


def _out_body(h_ref, w_ref, b_ref, o_ref):
    o_ref[...] = jnp.dot(h_ref[...], w_ref[...],
                         preferred_element_type=jnp.float32) + b_ref[...]


def _call(body, out_shapes):
    return pl.pallas_call(body, out_shape=out_shapes)


def _gates_cat(Wf, Wr):
    insz = Wf.shape[1]
    Wbig = jnp.zeros((2 * insz, 6 * H), jnp.float32)
    WfT, WrT = Wf.T, Wr.T
    for g in range(3):
        Wbig = Wbig.at[:insz, (2 * g) * H:(2 * g + 1) * H].set(
            WfT[:, g * H:(g + 1) * H])
        Wbig = Wbig.at[insz:, (2 * g + 1) * H:(2 * g + 2) * H].set(
            WrT[:, g * H:(g + 1) * H])
    return Wbig


def _bias_cat(bf, br):
    return jnp.concatenate([bf[0:H], br[0:H], bf[H:2 * H], br[H:2 * H],
                            bf[2 * H:], br[2 * H:]]).reshape(1, 6 * H)


def kernel(x, edge_index, W1, b1, W2, b2,
           gru_wih_l0, gru_whh_l0, gru_bih_l0, gru_bhh_l0,
           gru_wih_l0_r, gru_whh_l0_r, gru_bih_l0_r, gru_bhh_l0_r,
           gru_wih_l1, gru_whh_l1, gru_bih_l1, gru_bhh_l1,
           gru_wih_l1_r, gru_whh_l1_r, gru_bih_l1_r, gru_bhh_l1_r,
           Wlin, blin):
    f32 = jnp.float32
    src = edge_index[0]
    dst = edge_index[1]
    pad = EPAD - E
    src2d = jnp.concatenate([src, jnp.zeros((pad,), jnp.int32)]).reshape(-1, CH)
    dst2d = jnp.concatenate([dst, jnp.full((pad,), N, jnp.int32)]).reshape(-1, CH)

    ones16 = jnp.ones((CH, 16), f32)
    zeros16 = jnp.zeros((NACC, 16), f32)
    zeros32 = jnp.zeros((NACC, 32), f32)

    degp = _deg_kernel(dst2d, ones16, zeros16)
    degp2 = jnp.transpose(degp[:, :N, 0])

    y1, dinv = _call(_tc1_body, [jax.ShapeDtypeStruct((N, 16), f32),
                                 jax.ShapeDtypeStruct((N, 1), f32)])(
        x, W1, degp2)
    acc1 = _conv16(y1, src2d, dst2d, zeros16)
    y2 = _call(_tc2_body, jax.ShapeDtypeStruct((N, 32), f32))(
        acc1[0, :N], acc1[1, :N], y1, dinv, b1.reshape(1, 16), W2)

    acc2 = _conv32(y2, src2d, dst2d, zeros32)
    h = _call(_tc3_body, jax.ShapeDtypeStruct((N, 32), f32))(
        acc2[0, :N], acc2[1, :N], y2, dinv, b2.reshape(1, 32))

    haug = jnp.concatenate([h, h[::-1]], axis=-1)
    gi0 = _call(_gi_body, jax.ShapeDtypeStruct((N, 6 * H), f32))(
        haug, _gates_cat(gru_wih_l0, gru_wih_l0_r),
        _bias_cat(gru_bih_l0, gru_bih_l0_r))
    hall0 = _call(_scan_body, jax.ShapeDtypeStruct((N, 2 * H), f32))(
        gi0, _gates_cat(gru_whh_l0, gru_whh_l0_r),
        _bias_cat(gru_bhh_l0, gru_bhh_l0_r))

    h1 = jnp.concatenate([hall0[:, :H], hall0[::-1, H:]], axis=-1)
    h1aug = jnp.concatenate([h1, h1[::-1]], axis=-1)
    gi1 = _call(_gi_body, jax.ShapeDtypeStruct((N, 6 * H), f32))(
        h1aug, _gates_cat(gru_wih_l1, gru_wih_l1_r),
        _bias_cat(gru_bih_l1, gru_bih_l1_r))
    hall1 = _call(_scan_body, jax.ShapeDtypeStruct((N, 2 * H), f32))(
        gi1, _gates_cat(gru_whh_l1, gru_whh_l1_r),
        _bias_cat(gru_bhh_l1, gru_bhh_l1_r))

    h2cat = jnp.concatenate([hall1[:, :H], hall1[::-1, H:]], axis=-1)
    return _call(_out_body, jax.ShapeDtypeStruct((N, Wlin.shape[1]), f32))(
        h2cat, Wlin, blin.reshape(1, -1))


# speedup vs baseline: 5.8795x; 1.1188x over previous
import functools

import jax
import jax.numpy as jnp
from jax import lax
from jax.experimental import pallas as pl
from jax.experimental.pallas import tpu as pltpu
from jax.experimental.pallas import tpu_sc as plsc

N = 10000
E = 160000
H = 64

NC = 2
NS = 16
NW = NC * NS
CH = 128
EPW = 5120
NCHUNK = EPW // CH
EPAD = NW * EPW
NACC = 10112
ROWS = NACC // NS

_mesh = plsc.VectorSubcoreMesh(core_axis_name="c", subcore_axis_name="s")


def _make_deg_kernel():
    @functools.partial(
        pl.kernel,
        mesh=_mesh,
        compiler_params=pltpu.CompilerParams(use_tc_tiling_on_sc=False),
        out_type=jax.ShapeDtypeStruct((NC, NACC, 16), jnp.float32),
        scratch_types=[
            pltpu.VMEM((NCHUNK, CH), jnp.int32),
            pltpu.VMEM((CH, 16), jnp.float32),
            pltpu.VMEM((ROWS, 16), jnp.float32),
            pltpu.VMEM_SHARED((NACC, 16), jnp.float32),
        ],
    )
    def deg_kernel(dst2d, ones_hbm, zeros_hbm, out, idx_d, ones_v, stripe_v, acc_sh):
        cid = lax.axis_index("c")
        sid = lax.axis_index("s")
        wid = cid * NS + sid
        pltpu.sync_copy(zeros_hbm.at[pl.ds(sid * ROWS, ROWS)], stripe_v)
        pltpu.sync_copy(stripe_v, acc_sh.at[pl.ds(sid * ROWS, ROWS)])
        pltpu.sync_copy(ones_hbm, ones_v)
        pltpu.sync_copy(dst2d.at[pl.ds(wid * NCHUNK, NCHUNK)], idx_d)
        plsc.subcore_barrier()

        def body(k, carry):
            pltpu.sync_copy(ones_v, acc_sh.at[idx_d.at[k]], add=True)
            return carry

        lax.fori_loop(0, NCHUNK, body, 0)
        plsc.subcore_barrier()
        pltpu.sync_copy(acc_sh.at[pl.ds(sid * ROWS, ROWS)], stripe_v)
        pltpu.sync_copy(stripe_v, out.at[cid, pl.ds(sid * ROWS, ROWS)])

    return deg_kernel


def _make_conv_kernel(F):
    @functools.partial(
        pl.kernel,
        mesh=_mesh,
        compiler_params=pltpu.CompilerParams(use_tc_tiling_on_sc=False),
        out_type=jax.ShapeDtypeStruct((NC, NACC, F), jnp.float32),
        scratch_types=[
            pltpu.VMEM((NCHUNK, CH), jnp.int32),
            pltpu.VMEM((NCHUNK, CH), jnp.int32),
            pltpu.VMEM((CH, F), jnp.float32),
            pltpu.VMEM((ROWS, F), jnp.float32),
            pltpu.VMEM_SHARED((NACC, F), jnp.float32),
            pltpu.SemaphoreType.DMA,
        ],
    )
    def conv_kernel(y_hbm, src2d, dst2d, zeros_hbm, out,
                    idx_s, idx_d, rows_v, stripe_v, acc_sh, sem):
        cid = lax.axis_index("c")
        sid = lax.axis_index("s")
        wid = cid * NS + sid
        pltpu.sync_copy(zeros_hbm.at[pl.ds(sid * ROWS, ROWS)], stripe_v)
        pltpu.sync_copy(stripe_v, acc_sh.at[pl.ds(sid * ROWS, ROWS)])
        pltpu.sync_copy(src2d.at[pl.ds(wid * NCHUNK, NCHUNK)], idx_s)
        pltpu.sync_copy(dst2d.at[pl.ds(wid * NCHUNK, NCHUNK)], idx_d)
        plsc.subcore_barrier()

        def body(k, carry):
            pltpu.async_copy(y_hbm.at[idx_s.at[k]], rows_v, sem).wait()
            pltpu.sync_copy(rows_v, acc_sh.at[idx_d.at[k]], add=True)
            return carry

        lax.fori_loop(0, NCHUNK, body, 0)
        plsc.subcore_barrier()
        pltpu.sync_copy(acc_sh.at[pl.ds(sid * ROWS, ROWS)], stripe_v)
        pltpu.sync_copy(stripe_v, out.at[cid, pl.ds(sid * ROWS, ROWS)])

    return conv_kernel


_deg_kernel = _make_deg_kernel()
_conv16 = _make_conv_kernel(16)
_conv32 = _make_conv_kernel(32)


def _tc1_body(x_ref, w1_ref, degp_ref, y1_ref, dinv_ref):
    deg = degp_ref[:, 0:1] + degp_ref[:, 1:2] + 1.0
    dinv = lax.rsqrt(deg)
    xw = jnp.dot(x_ref[...], w1_ref[...], preferred_element_type=jnp.float32)
    y1_ref[...] = dinv * xw
    dinv_ref[...] = dinv


def _tc2_body(acca_ref, accb_ref, y_ref, dinv_ref, b_ref, w2_ref, y2_ref):
    dinv = dinv_ref[...]
    h = acca_ref[...] + accb_ref[...] + y_ref[...]
    h = jax.nn.relu(dinv * h + b_ref[...])
    y2_ref[...] = dinv * jnp.dot(h, w2_ref[...], preferred_element_type=jnp.float32)


def _tc3_body(acca_ref, accb_ref, y_ref, dinv_ref, b_ref, h_ref):
    h = acca_ref[...] + accb_ref[...] + y_ref[...]
    h_ref[...] = jax.nn.relu(dinv_ref[...] * h + b_ref[...])


def _gi_body(haug_ref, wbig_ref, bcat_ref, gi_ref):
    gi_ref[...] = jnp.dot(haug_ref[...], wbig_ref[...],
                          preferred_element_type=jnp.float32) + bcat_ref[...]


def _scan_body(gi_ref, wblk_ref, bhh_ref, hall_ref):
    bhh = bhh_ref[...]

    def step(t, carry):
        h, hbt = carry
        gi_t = gi_ref[pl.ds(t, 1), :]
        gh0 = jnp.sum(hbt * wblk_ref[:, 0:128], axis=0, keepdims=True)
        gh1 = jnp.sum(hbt * wblk_ref[:, 128:256], axis=0, keepdims=True)
        gh2 = jnp.sum(hbt * wblk_ref[:, 256:384], axis=0, keepdims=True)
        gh0 = gh0 + bhh[:, 0:128]
        gh1 = gh1 + bhh[:, 128:256]
        gh2 = gh2 + bhh[:, 256:384]
        r = 0.5 + 0.5 * jnp.tanh(0.5 * (gi_t[:, 0:128] + gh0))
        z = 0.5 + 0.5 * jnp.tanh(0.5 * (gi_t[:, 128:256] + gh1))
        cand = jnp.tanh(gi_t[:, 256:384] + r * gh2)
        hn = (1.0 - z) * cand + z * h
        hall_ref[pl.ds(t, 1), :] = hn
        hbt_n = jnp.broadcast_to(hn, (128, 128)).T
        return (hn, hbt_n)

    lax.fori_loop(0, N, step, (jnp.zeros((1, 2 * H), jnp.float32),
                               jnp.zeros((128, 128), jnp.float32)),
                  unroll=2)


def _out_body(h_ref, w_ref, b_ref, o_ref):
    o_ref[...] = jnp.dot(h_ref[...], w_ref[...],
                         preferred_element_type=jnp.float32) + b_ref[...]


def _call(body, out_shapes):
    return pl.pallas_call(body, out_shape=out_shapes)


def _gates_cat(Wf, Wr):
    insz = Wf.shape[1]
    Wbig = jnp.zeros((2 * insz, 6 * H), jnp.float32)
    WfT, WrT = Wf.T, Wr.T
    for g in range(3):
        Wbig = Wbig.at[:insz, (2 * g) * H:(2 * g + 1) * H].set(
            WfT[:, g * H:(g + 1) * H])
        Wbig = Wbig.at[insz:, (2 * g + 1) * H:(2 * g + 2) * H].set(
            WrT[:, g * H:(g + 1) * H])
    return Wbig


def _bias_cat(bf, br):
    return jnp.concatenate([bf[0:H], br[0:H], bf[H:2 * H], br[H:2 * H],
                            bf[2 * H:], br[2 * H:]]).reshape(1, 6 * H)


def kernel(x, edge_index, W1, b1, W2, b2,
           gru_wih_l0, gru_whh_l0, gru_bih_l0, gru_bhh_l0,
           gru_wih_l0_r, gru_whh_l0_r, gru_bih_l0_r, gru_bhh_l0_r,
           gru_wih_l1, gru_whh_l1, gru_bih_l1, gru_bhh_l1,
           gru_wih_l1_r, gru_whh_l1_r, gru_bih_l1_r, gru_bhh_l1_r,
           Wlin, blin):
    f32 = jnp.float32
    src = edge_index[0]
    dst = edge_index[1]
    pad = EPAD - E
    src2d = jnp.concatenate([src, jnp.zeros((pad,), jnp.int32)]).reshape(-1, CH)
    dst2d = jnp.concatenate([dst, jnp.full((pad,), N, jnp.int32)]).reshape(-1, CH)

    ones16 = jnp.ones((CH, 16), f32)
    zeros16 = jnp.zeros((NACC, 16), f32)
    zeros32 = jnp.zeros((NACC, 32), f32)

    degp = _deg_kernel(dst2d, ones16, zeros16)
    degp2 = jnp.transpose(degp[:, :N, 0])

    y1, dinv = _call(_tc1_body, [jax.ShapeDtypeStruct((N, 16), f32),
                                 jax.ShapeDtypeStruct((N, 1), f32)])(
        x, W1, degp2)
    acc1 = _conv16(y1, src2d, dst2d, zeros16)
    y2 = _call(_tc2_body, jax.ShapeDtypeStruct((N, 32), f32))(
        acc1[0, :N], acc1[1, :N], y1, dinv, b1.reshape(1, 16), W2)

    acc2 = _conv32(y2, src2d, dst2d, zeros32)
    h = _call(_tc3_body, jax.ShapeDtypeStruct((N, 32), f32))(
        acc2[0, :N], acc2[1, :N], y2, dinv, b2.reshape(1, 32))

    haug = jnp.concatenate([h, h[::-1]], axis=-1)
    gi0 = _call(_gi_body, jax.ShapeDtypeStruct((N, 6 * H), f32))(
        haug, _gates_cat(gru_wih_l0, gru_wih_l0_r),
        _bias_cat(gru_bih_l0, gru_bih_l0_r))
    hall0 = _call(_scan_body, jax.ShapeDtypeStruct((N, 2 * H), f32))(
        gi0, _gates_cat(gru_whh_l0, gru_whh_l0_r),
        _bias_cat(gru_bhh_l0, gru_bhh_l0_r))

    h1 = jnp.concatenate([hall0[:, :H], hall0[::-1, H:]], axis=-1)
    h1aug = jnp.concatenate([h1, h1[::-1]], axis=-1)
    gi1 = _call(_gi_body, jax.ShapeDtypeStruct((N, 6 * H), f32))(
        h1aug, _gates_cat(gru_wih_l1, gru_wih_l1_r),
        _bias_cat(gru_bih_l1, gru_bih_l1_r))
    hall1 = _call(_scan_body, jax.ShapeDtypeStruct((N, 2 * H), f32))(
        gi1, _gates_cat(gru_whh_l1, gru_whh_l1_r),
        _bias_cat(gru_bhh_l1, gru_bhh_l1_r))

    h2cat = jnp.concatenate([hall1[:, :H], hall1[::-1, H:]], axis=-1)
    return _call(_out_body, jax.ShapeDtypeStruct((N, Wlin.shape[1]), f32))(
        h2cat, Wlin, blin.reshape(1, -1))


# speedup vs baseline: 5.9541x; 1.0127x over previous
import functools

import jax
import jax.numpy as jnp
from jax import lax
from jax.experimental import pallas as pl
from jax.experimental.pallas import tpu as pltpu
from jax.experimental.pallas import tpu_sc as plsc

N = 10000
E = 160000
H = 64

NC = 2
NS = 16
NW = NC * NS
CH = 128
EPW = 5120
NCHUNK = EPW // CH
EPAD = NW * EPW
NACC = 10112
ROWS = NACC // NS

_mesh = plsc.VectorSubcoreMesh(core_axis_name="c", subcore_axis_name="s")


def _make_deg_kernel():
    @functools.partial(
        pl.kernel,
        mesh=_mesh,
        compiler_params=pltpu.CompilerParams(use_tc_tiling_on_sc=False),
        out_type=jax.ShapeDtypeStruct((NC, NACC, 16), jnp.float32),
        scratch_types=[
            pltpu.VMEM((NCHUNK, CH), jnp.int32),
            pltpu.VMEM((CH, 16), jnp.float32),
            pltpu.VMEM((ROWS, 16), jnp.float32),
            pltpu.VMEM_SHARED((NACC, 16), jnp.float32),
        ],
    )
    def deg_kernel(dst2d, ones_hbm, zeros_hbm, out, idx_d, ones_v, stripe_v, acc_sh):
        cid = lax.axis_index("c")
        sid = lax.axis_index("s")
        wid = cid * NS + sid
        pltpu.sync_copy(zeros_hbm.at[pl.ds(sid * ROWS, ROWS)], stripe_v)
        pltpu.sync_copy(stripe_v, acc_sh.at[pl.ds(sid * ROWS, ROWS)])
        pltpu.sync_copy(ones_hbm, ones_v)
        pltpu.sync_copy(dst2d.at[pl.ds(wid * NCHUNK, NCHUNK)], idx_d)
        plsc.subcore_barrier()

        def body(k, carry):
            pltpu.sync_copy(ones_v, acc_sh.at[idx_d.at[k]], add=True)
            return carry

        lax.fori_loop(0, NCHUNK, body, 0)
        plsc.subcore_barrier()
        pltpu.sync_copy(acc_sh.at[pl.ds(sid * ROWS, ROWS)], stripe_v)
        pltpu.sync_copy(stripe_v, out.at[cid, pl.ds(sid * ROWS, ROWS)])

    return deg_kernel


def _make_conv_kernel(F):
    @functools.partial(
        pl.kernel,
        mesh=_mesh,
        compiler_params=pltpu.CompilerParams(use_tc_tiling_on_sc=False),
        out_type=jax.ShapeDtypeStruct((NC, NACC, F), jnp.float32),
        scratch_types=[
            pltpu.VMEM((NCHUNK, CH), jnp.int32),
            pltpu.VMEM((NCHUNK, CH), jnp.int32),
            pltpu.VMEM((CH, F), jnp.float32),
            pltpu.VMEM((CH, F), jnp.float32),
            pltpu.VMEM((ROWS, F), jnp.float32),
            pltpu.VMEM_SHARED((NACC, F), jnp.float32),
            pltpu.SemaphoreType.DMA,
            pltpu.SemaphoreType.DMA,
        ],
    )
    def conv_kernel(y_hbm, src2d, dst2d, zeros_hbm, out,
                    idx_s, idx_d, rows_a, rows_b, stripe_v, acc_sh,
                    sem_a, sem_b):
        cid = lax.axis_index("c")
        sid = lax.axis_index("s")
        wid = cid * NS + sid
        pltpu.sync_copy(zeros_hbm.at[pl.ds(sid * ROWS, ROWS)], stripe_v)
        pltpu.sync_copy(stripe_v, acc_sh.at[pl.ds(sid * ROWS, ROWS)])
        pltpu.sync_copy(src2d.at[pl.ds(wid * NCHUNK, NCHUNK)], idx_s)
        pltpu.sync_copy(dst2d.at[pl.ds(wid * NCHUNK, NCHUNK)], idx_d)
        plsc.subcore_barrier()

        ga = pltpu.async_copy(y_hbm.at[idx_s.at[0]], rows_a, sem_a)

        def body(i, carry):
            k0 = 2 * i
            gb = pltpu.async_copy(y_hbm.at[idx_s.at[k0 + 1]], rows_b, sem_b)
            pltpu.make_async_copy(y_hbm.at[idx_s.at[k0]], rows_a, sem_a).wait()
            pltpu.sync_copy(rows_a, acc_sh.at[idx_d.at[k0]], add=True)
            knext = jnp.minimum(k0 + 2, NCHUNK - 1)
            pltpu.async_copy(y_hbm.at[idx_s.at[knext]], rows_a, sem_a)
            gb.wait()
            pltpu.sync_copy(rows_b, acc_sh.at[idx_d.at[k0 + 1]], add=True)
            return carry

        lax.fori_loop(0, NCHUNK // 2, body, 0)
        pltpu.make_async_copy(y_hbm.at[idx_s.at[NCHUNK - 1]], rows_a,
                              sem_a).wait()
        plsc.subcore_barrier()
        pltpu.sync_copy(acc_sh.at[pl.ds(sid * ROWS, ROWS)], stripe_v)
        pltpu.sync_copy(stripe_v, out.at[cid, pl.ds(sid * ROWS, ROWS)])

    return conv_kernel


_deg_kernel = _make_deg_kernel()
_conv16 = _make_conv_kernel(16)
_conv32 = _make_conv_kernel(32)


def _tc1_body(x_ref, w1_ref, degp_ref, y1_ref, dinv_ref):
    deg = degp_ref[:, 0:1] + degp_ref[:, 1:2] + 1.0
    dinv = lax.rsqrt(deg)
    xw = jnp.dot(x_ref[...], w1_ref[...], preferred_element_type=jnp.float32)
    y1_ref[...] = dinv * xw
    dinv_ref[...] = dinv


def _tc2_body(acca_ref, accb_ref, y_ref, dinv_ref, b_ref, w2_ref, y2_ref):
    dinv = dinv_ref[...]
    h = acca_ref[...] + accb_ref[...] + y_ref[...]
    h = jax.nn.relu(dinv * h + b_ref[...])
    y2_ref[...] = dinv * jnp.dot(h, w2_ref[...], preferred_element_type=jnp.float32)


def _tc3_body(acca_ref, accb_ref, y_ref, dinv_ref, b_ref, h_ref):
    h = acca_ref[...] + accb_ref[...] + y_ref[...]
    h_ref[...] = jax.nn.relu(dinv_ref[...] * h + b_ref[...])


def _gi_body(haug_ref, wbig_ref, bcat_ref, gi_ref):
    gi_ref[...] = jnp.dot(haug_ref[...], wbig_ref[...],
                          preferred_element_type=jnp.float32) + bcat_ref[...]


def _scan_body(gi_ref, wblk_ref, bhh_ref, hall_ref):
    bhh = bhh_ref[...]

    def step(t, carry):
        h, hbt = carry
        gi_t = gi_ref[pl.ds(t, 1), :]
        gh0 = jnp.sum(hbt * wblk_ref[:, 0:128], axis=0, keepdims=True)
        gh1 = jnp.sum(hbt * wblk_ref[:, 128:256], axis=0, keepdims=True)
        gh2 = jnp.sum(hbt * wblk_ref[:, 256:384], axis=0, keepdims=True)
        gh0 = gh0 + bhh[:, 0:128]
        gh1 = gh1 + bhh[:, 128:256]
        gh2 = gh2 + bhh[:, 256:384]
        r = 0.5 + 0.5 * jnp.tanh(0.5 * (gi_t[:, 0:128] + gh0))
        z = 0.5 + 0.5 * jnp.tanh(0.5 * (gi_t[:, 128:256] + gh1))
        cand = jnp.tanh(gi_t[:, 256:384] + r * gh2)
        hn = (1.0 - z) * cand + z * h
        hall_ref[pl.ds(t, 1), :] = hn
        hbt_n = jnp.broadcast_to(hn, (128, 128)).T
        return (hn, hbt_n)

    lax.fori_loop(0, N, step, (jnp.zeros((1, 2 * H), jnp.float32),
                               jnp.zeros((128, 128), jnp.float32)),
                  unroll=2)


def _out_body(h_ref, w_ref, b_ref, o_ref):
    o_ref[...] = jnp.dot(h_ref[...], w_ref[...],
                         preferred_element_type=jnp.float32) + b_ref[...]


def _call(body, out_shapes):
    return pl.pallas_call(body, out_shape=out_shapes)


def _gates_cat(Wf, Wr):
    insz = Wf.shape[1]
    Wbig = jnp.zeros((2 * insz, 6 * H), jnp.float32)
    WfT, WrT = Wf.T, Wr.T
    for g in range(3):
        Wbig = Wbig.at[:insz, (2 * g) * H:(2 * g + 1) * H].set(
            WfT[:, g * H:(g + 1) * H])
        Wbig = Wbig.at[insz:, (2 * g + 1) * H:(2 * g + 2) * H].set(
            WrT[:, g * H:(g + 1) * H])
    return Wbig


def _bias_cat(bf, br):
    return jnp.concatenate([bf[0:H], br[0:H], bf[H:2 * H], br[H:2 * H],
                            bf[2 * H:], br[2 * H:]]).reshape(1, 6 * H)


def kernel(x, edge_index, W1, b1, W2, b2,
           gru_wih_l0, gru_whh_l0, gru_bih_l0, gru_bhh_l0,
           gru_wih_l0_r, gru_whh_l0_r, gru_bih_l0_r, gru_bhh_l0_r,
           gru_wih_l1, gru_whh_l1, gru_bih_l1, gru_bhh_l1,
           gru_wih_l1_r, gru_whh_l1_r, gru_bih_l1_r, gru_bhh_l1_r,
           Wlin, blin):
    f32 = jnp.float32
    src = edge_index[0]
    dst = edge_index[1]
    pad = EPAD - E
    src2d = jnp.concatenate([src, jnp.zeros((pad,), jnp.int32)]).reshape(-1, CH)
    dst2d = jnp.concatenate([dst, jnp.full((pad,), N, jnp.int32)]).reshape(-1, CH)

    ones16 = jnp.ones((CH, 16), f32)
    zeros16 = jnp.zeros((NACC, 16), f32)
    zeros32 = jnp.zeros((NACC, 32), f32)

    degp = _deg_kernel(dst2d, ones16, zeros16)
    degp2 = jnp.transpose(degp[:, :N, 0])

    y1, dinv = _call(_tc1_body, [jax.ShapeDtypeStruct((N, 16), f32),
                                 jax.ShapeDtypeStruct((N, 1), f32)])(
        x, W1, degp2)
    acc1 = _conv16(y1, src2d, dst2d, zeros16)
    y2 = _call(_tc2_body, jax.ShapeDtypeStruct((N, 32), f32))(
        acc1[0, :N], acc1[1, :N], y1, dinv, b1.reshape(1, 16), W2)

    acc2 = _conv32(y2, src2d, dst2d, zeros32)
    h = _call(_tc3_body, jax.ShapeDtypeStruct((N, 32), f32))(
        acc2[0, :N], acc2[1, :N], y2, dinv, b2.reshape(1, 32))

    haug = jnp.concatenate([h, h[::-1]], axis=-1)
    gi0 = _call(_gi_body, jax.ShapeDtypeStruct((N, 6 * H), f32))(
        haug, _gates_cat(gru_wih_l0, gru_wih_l0_r),
        _bias_cat(gru_bih_l0, gru_bih_l0_r))
    hall0 = _call(_scan_body, jax.ShapeDtypeStruct((N, 2 * H), f32))(
        gi0, _gates_cat(gru_whh_l0, gru_whh_l0_r),
        _bias_cat(gru_bhh_l0, gru_bhh_l0_r))

    h1 = jnp.concatenate([hall0[:, :H], hall0[::-1, H:]], axis=-1)
    h1aug = jnp.concatenate([h1, h1[::-1]], axis=-1)
    gi1 = _call(_gi_body, jax.ShapeDtypeStruct((N, 6 * H), f32))(
        h1aug, _gates_cat(gru_wih_l1, gru_wih_l1_r),
        _bias_cat(gru_bih_l1, gru_bih_l1_r))
    hall1 = _call(_scan_body, jax.ShapeDtypeStruct((N, 2 * H), f32))(
        gi1, _gates_cat(gru_whh_l1, gru_whh_l1_r),
        _bias_cat(gru_bhh_l1, gru_bhh_l1_r))

    h2cat = jnp.concatenate([hall1[:, :H], hall1[::-1, H:]], axis=-1)
    return _call(_out_body, jax.ShapeDtypeStruct((N, Wlin.shape[1]), f32))(
        h2cat, Wlin, blin.reshape(1, -1))


# speedup vs baseline: 6.0135x; 1.0100x over previous
import functools

import jax
import jax.numpy as jnp
from jax import lax
from jax.experimental import pallas as pl
from jax.experimental.pallas import tpu as pltpu
from jax.experimental.pallas import tpu_sc as plsc

N = 10000
E = 160000
H = 64

NC = 2
NS = 16
NW = NC * NS
CH = 128
EPW = 5120
NCHUNK = EPW // CH
EPAD = NW * EPW
NACC = 10112
ROWS = NACC // NS

_mesh = plsc.VectorSubcoreMesh(core_axis_name="c", subcore_axis_name="s")


def _make_deg_kernel():
    @functools.partial(
        pl.kernel,
        mesh=_mesh,
        compiler_params=pltpu.CompilerParams(use_tc_tiling_on_sc=False),
        out_type=jax.ShapeDtypeStruct((NC, NACC, 16), jnp.float32),
        scratch_types=[
            pltpu.VMEM((NCHUNK, CH), jnp.int32),
            pltpu.VMEM((CH, 16), jnp.float32),
            pltpu.VMEM((ROWS, 16), jnp.float32),
            pltpu.VMEM_SHARED((NACC, 16), jnp.float32),
        ],
    )
    def deg_kernel(dst2d, ones_hbm, zeros_hbm, out, idx_d, ones_v, stripe_v, acc_sh):
        cid = lax.axis_index("c")
        sid = lax.axis_index("s")
        wid = cid * NS + sid
        pltpu.sync_copy(zeros_hbm.at[pl.ds(sid * ROWS, ROWS)], stripe_v)
        pltpu.sync_copy(stripe_v, acc_sh.at[pl.ds(sid * ROWS, ROWS)])
        pltpu.sync_copy(ones_hbm, ones_v)
        pltpu.sync_copy(dst2d.at[pl.ds(wid * NCHUNK, NCHUNK)], idx_d)
        plsc.subcore_barrier()

        def body(k, carry):
            pltpu.sync_copy(ones_v, acc_sh.at[idx_d.at[k]], add=True)
            return carry

        lax.fori_loop(0, NCHUNK, body, 0)
        plsc.subcore_barrier()
        pltpu.sync_copy(acc_sh.at[pl.ds(sid * ROWS, ROWS)], stripe_v)
        pltpu.sync_copy(stripe_v, out.at[cid, pl.ds(sid * ROWS, ROWS)])

    return deg_kernel


def _make_conv_kernel(F):
    @functools.partial(
        pl.kernel,
        mesh=_mesh,
        compiler_params=pltpu.CompilerParams(use_tc_tiling_on_sc=False),
        out_type=jax.ShapeDtypeStruct((NC, NACC, F), jnp.float32),
        scratch_types=[
            pltpu.VMEM((NCHUNK, CH), jnp.int32),
            pltpu.VMEM((NCHUNK, CH), jnp.int32),
            pltpu.VMEM((CH, F), jnp.float32),
            pltpu.VMEM((CH, F), jnp.float32),
            pltpu.VMEM((ROWS, F), jnp.float32),
            pltpu.VMEM_SHARED((NACC, F), jnp.float32),
            pltpu.SemaphoreType.DMA,
            pltpu.SemaphoreType.DMA,
        ],
    )
    def conv_kernel(y_hbm, src2d, dst2d, zeros_hbm, out,
                    idx_s, idx_d, rows_a, rows_b, stripe_v, acc_sh,
                    sem_a, sem_b):
        cid = lax.axis_index("c")
        sid = lax.axis_index("s")
        wid = cid * NS + sid
        pltpu.sync_copy(zeros_hbm.at[pl.ds(sid * ROWS, ROWS)], stripe_v)
        pltpu.sync_copy(stripe_v, acc_sh.at[pl.ds(sid * ROWS, ROWS)])
        pltpu.sync_copy(src2d.at[pl.ds(wid * NCHUNK, NCHUNK)], idx_s)
        pltpu.sync_copy(dst2d.at[pl.ds(wid * NCHUNK, NCHUNK)], idx_d)
        plsc.subcore_barrier()

        ga = pltpu.async_copy(y_hbm.at[idx_s.at[0]], rows_a, sem_a)

        def body(i, carry):
            k0 = 2 * i
            gb = pltpu.async_copy(y_hbm.at[idx_s.at[k0 + 1]], rows_b, sem_b)
            pltpu.make_async_copy(y_hbm.at[idx_s.at[k0]], rows_a, sem_a).wait()
            pltpu.sync_copy(rows_a, acc_sh.at[idx_d.at[k0]], add=True)
            knext = jnp.minimum(k0 + 2, NCHUNK - 1)
            pltpu.async_copy(y_hbm.at[idx_s.at[knext]], rows_a, sem_a)
            gb.wait()
            pltpu.sync_copy(rows_b, acc_sh.at[idx_d.at[k0 + 1]], add=True)
            return carry

        lax.fori_loop(0, NCHUNK // 2, body, 0)
        pltpu.make_async_copy(y_hbm.at[idx_s.at[NCHUNK - 1]], rows_a,
                              sem_a).wait()
        plsc.subcore_barrier()
        pltpu.sync_copy(acc_sh.at[pl.ds(sid * ROWS, ROWS)], stripe_v)
        pltpu.sync_copy(stripe_v, out.at[cid, pl.ds(sid * ROWS, ROWS)])

    return conv_kernel


_deg_kernel = _make_deg_kernel()
_conv16 = _make_conv_kernel(16)
_conv32 = _make_conv_kernel(32)


def _tc1_body(x_ref, w1_ref, degp_ref, y1_ref, dinv_ref):
    deg = degp_ref[:, 0:1] + degp_ref[:, 1:2] + 1.0
    dinv = lax.rsqrt(deg)
    xw = jnp.dot(x_ref[...], w1_ref[...], preferred_element_type=jnp.float32)
    y1_ref[...] = dinv * xw
    dinv_ref[...] = dinv


def _tc2_body(acca_ref, accb_ref, y_ref, dinv_ref, b_ref, w2_ref, y2_ref):
    dinv = dinv_ref[...]
    h = acca_ref[...] + accb_ref[...] + y_ref[...]
    h = jax.nn.relu(dinv * h + b_ref[...])
    y2_ref[...] = dinv * jnp.dot(h, w2_ref[...], preferred_element_type=jnp.float32)


def _tc3_body(acca_ref, accb_ref, y_ref, dinv_ref, b_ref, h_ref):
    h = acca_ref[...] + accb_ref[...] + y_ref[...]
    h_ref[...] = jax.nn.relu(dinv_ref[...] * h + b_ref[...])


def _gi_body(haug_ref, wbig_ref, bcat_ref, gi_ref):
    gi_ref[...] = jnp.dot(haug_ref[...], wbig_ref[...],
                          preferred_element_type=jnp.float32) + bcat_ref[...]


def _scan_body(gi_ref, wc_ref, bhn_ref, hall_ref):
    bhn = bhn_ref[...]

    def step(t, carry):
        h, hbt = carry
        gi_t = gi_ref[pl.ds(t, 1), :]
        s0 = jnp.sum(hbt * wc_ref[:, 0:128], axis=0, keepdims=True)
        s1 = jnp.sum(hbt * wc_ref[:, 128:256], axis=0, keepdims=True)
        s2h = jnp.sum(hbt * wc_ref[:, 256:384], axis=0, keepdims=True) + bhn
        tr = jnp.tanh(gi_t[:, 0:128] + s0)
        tz = jnp.tanh(gi_t[:, 128:256] + s1)
        c = jnp.tanh(gi_t[:, 256:384] + s2h + tr * s2h)
        hn = 0.5 * (c + h) + tz * (0.5 * (h - c))
        hall_ref[pl.ds(t, 1), :] = hn
        hbt_n = jnp.broadcast_to(hn, (128, 128)).T
        return (hn, hbt_n)

    lax.fori_loop(0, N, step, (jnp.zeros((1, 2 * H), jnp.float32),
                               jnp.zeros((128, 128), jnp.float32)),
                  unroll=2)


def _out_body(h_ref, w_ref, b_ref, o_ref):
    o_ref[...] = jnp.dot(h_ref[...], w_ref[...],
                         preferred_element_type=jnp.float32) + b_ref[...]


def _call(body, out_shapes):
    return pl.pallas_call(body, out_shape=out_shapes)


def _gates_cat(Wf, Wr):
    insz = Wf.shape[1]
    Wbig = jnp.zeros((2 * insz, 6 * H), jnp.float32)
    WfT, WrT = Wf.T, Wr.T
    for g in range(3):
        Wbig = Wbig.at[:insz, (2 * g) * H:(2 * g + 1) * H].set(
            WfT[:, g * H:(g + 1) * H])
        Wbig = Wbig.at[insz:, (2 * g + 1) * H:(2 * g + 2) * H].set(
            WrT[:, g * H:(g + 1) * H])
    return Wbig


def _bias_cat(bf, br):
    return jnp.concatenate([bf[0:H], br[0:H], bf[H:2 * H], br[H:2 * H],
                            bf[2 * H:], br[2 * H:]]).reshape(1, 6 * H)


def _scan_prep(wih_f, wih_r, bih_f, bih_r, whh_f, whh_r, bhh_f, bhh_r):
    half = jnp.concatenate([jnp.full((256,), 0.5, jnp.float32),
                            jnp.ones((128,), jnp.float32)]).reshape(1, 384)
    wbig = _gates_cat(wih_f, wih_r) * half
    bi = _bias_cat(bih_f, bih_r)
    bh = _bias_cat(bhh_f, bhh_r)
    gi_bias = jnp.concatenate([0.5 * (bi[:, 0:256] + bh[:, 0:256]),
                               bi[:, 256:384]], axis=1)
    wblk = _gates_cat(whh_f, whh_r)
    wc = 0.5 * wblk
    bhn_half = 0.5 * bh[:, 256:384]
    return wbig, gi_bias, wc, bhn_half


def kernel(x, edge_index, W1, b1, W2, b2,
           gru_wih_l0, gru_whh_l0, gru_bih_l0, gru_bhh_l0,
           gru_wih_l0_r, gru_whh_l0_r, gru_bih_l0_r, gru_bhh_l0_r,
           gru_wih_l1, gru_whh_l1, gru_bih_l1, gru_bhh_l1,
           gru_wih_l1_r, gru_whh_l1_r, gru_bih_l1_r, gru_bhh_l1_r,
           Wlin, blin):
    f32 = jnp.float32
    src = edge_index[0]
    dst = edge_index[1]
    pad = EPAD - E
    src2d = jnp.concatenate([src, jnp.zeros((pad,), jnp.int32)]).reshape(-1, CH)
    dst2d = jnp.concatenate([dst, jnp.full((pad,), N, jnp.int32)]).reshape(-1, CH)

    ones16 = jnp.ones((CH, 16), f32)
    zeros16 = jnp.zeros((NACC, 16), f32)
    zeros32 = jnp.zeros((NACC, 32), f32)

    degp = _deg_kernel(dst2d, ones16, zeros16)
    degp2 = jnp.transpose(degp[:, :N, 0])

    y1, dinv = _call(_tc1_body, [jax.ShapeDtypeStruct((N, 16), f32),
                                 jax.ShapeDtypeStruct((N, 1), f32)])(
        x, W1, degp2)
    acc1 = _conv16(y1, src2d, dst2d, zeros16)
    y2 = _call(_tc2_body, jax.ShapeDtypeStruct((N, 32), f32))(
        acc1[0, :N], acc1[1, :N], y1, dinv, b1.reshape(1, 16), W2)

    acc2 = _conv32(y2, src2d, dst2d, zeros32)
    h = _call(_tc3_body, jax.ShapeDtypeStruct((N, 32), f32))(
        acc2[0, :N], acc2[1, :N], y2, dinv, b2.reshape(1, 32))

    haug = jnp.concatenate([h, h[::-1]], axis=-1)
    wbig0, gib0, wc0, bhn0 = _scan_prep(
        gru_wih_l0, gru_wih_l0_r, gru_bih_l0, gru_bih_l0_r,
        gru_whh_l0, gru_whh_l0_r, gru_bhh_l0, gru_bhh_l0_r)
    gi0 = _call(_gi_body, jax.ShapeDtypeStruct((N, 6 * H), f32))(
        haug, wbig0, gib0)
    hall0 = _call(_scan_body, jax.ShapeDtypeStruct((N, 2 * H), f32))(
        gi0, wc0, bhn0)

    h1 = jnp.concatenate([hall0[:, :H], hall0[::-1, H:]], axis=-1)
    h1aug = jnp.concatenate([h1, h1[::-1]], axis=-1)
    wbig1, gib1, wc1, bhn1 = _scan_prep(
        gru_wih_l1, gru_wih_l1_r, gru_bih_l1, gru_bih_l1_r,
        gru_whh_l1, gru_whh_l1_r, gru_bhh_l1, gru_bhh_l1_r)
    gi1 = _call(_gi_body, jax.ShapeDtypeStruct((N, 6 * H), f32))(
        h1aug, wbig1, gib1)
    hall1 = _call(_scan_body, jax.ShapeDtypeStruct((N, 2 * H), f32))(
        gi1, wc1, bhn1)

    h2cat = jnp.concatenate([hall1[:, :H], hall1[::-1, H:]], axis=-1)
    return _call(_out_body, jax.ShapeDtypeStruct((N, Wlin.shape[1]), f32))(
        h2cat, Wlin, blin.reshape(1, -1))


# speedup vs baseline: 6.1632x; 1.0249x over previous
import functools

import jax
import jax.numpy as jnp
from jax import lax
from jax.experimental import pallas as pl
from jax.experimental.pallas import tpu as pltpu
from jax.experimental.pallas import tpu_sc as plsc

N = 10000
E = 160000
H = 64

NC = 2
NS = 16
NW = NC * NS
CH = 128
EPW = 5120
NCHUNK = EPW // CH
EPAD = NW * EPW
NACC = 10112
ROWS = NACC // NS

_mesh = plsc.VectorSubcoreMesh(core_axis_name="c", subcore_axis_name="s")


def _make_deg_kernel():
    @functools.partial(
        pl.kernel,
        mesh=_mesh,
        compiler_params=pltpu.CompilerParams(use_tc_tiling_on_sc=False),
        out_type=jax.ShapeDtypeStruct((NC, NACC, 16), jnp.float32),
        scratch_types=[
            pltpu.VMEM((NCHUNK, CH), jnp.int32),
            pltpu.VMEM((CH, 16), jnp.float32),
            pltpu.VMEM((ROWS, 16), jnp.float32),
            pltpu.VMEM_SHARED((NACC, 16), jnp.float32),
        ],
    )
    def deg_kernel(dst2d, ones_hbm, zeros_hbm, out, idx_d, ones_v, stripe_v, acc_sh):
        cid = lax.axis_index("c")
        sid = lax.axis_index("s")
        wid = cid * NS + sid
        pltpu.sync_copy(zeros_hbm.at[pl.ds(sid * ROWS, ROWS)], stripe_v)
        pltpu.sync_copy(stripe_v, acc_sh.at[pl.ds(sid * ROWS, ROWS)])
        pltpu.sync_copy(ones_hbm, ones_v)
        pltpu.sync_copy(dst2d.at[pl.ds(wid * NCHUNK, NCHUNK)], idx_d)
        plsc.subcore_barrier()

        def body(k, carry):
            pltpu.sync_copy(ones_v, acc_sh.at[idx_d.at[k]], add=True)
            return carry

        lax.fori_loop(0, NCHUNK, body, 0)
        plsc.subcore_barrier()
        pltpu.sync_copy(acc_sh.at[pl.ds(sid * ROWS, ROWS)], stripe_v)
        pltpu.sync_copy(stripe_v, out.at[cid, pl.ds(sid * ROWS, ROWS)])

    return deg_kernel


def _make_conv_kernel(F):
    @functools.partial(
        pl.kernel,
        mesh=_mesh,
        compiler_params=pltpu.CompilerParams(use_tc_tiling_on_sc=False),
        out_type=jax.ShapeDtypeStruct((NC, NACC, F), jnp.float32),
        scratch_types=[
            pltpu.VMEM((NCHUNK, CH), jnp.int32),
            pltpu.VMEM((NCHUNK, CH), jnp.int32),
            pltpu.VMEM((CH, F), jnp.float32),
            pltpu.VMEM((CH, F), jnp.float32),
            pltpu.VMEM((ROWS, F), jnp.float32),
            pltpu.VMEM_SHARED((NACC, F), jnp.float32),
            pltpu.SemaphoreType.DMA,
            pltpu.SemaphoreType.DMA,
        ],
    )
    def conv_kernel(y_hbm, src2d, dst2d, zeros_hbm, out,
                    idx_s, idx_d, rows_a, rows_b, stripe_v, acc_sh,
                    sem_a, sem_b):
        cid = lax.axis_index("c")
        sid = lax.axis_index("s")
        wid = cid * NS + sid
        pltpu.sync_copy(zeros_hbm.at[pl.ds(sid * ROWS, ROWS)], stripe_v)
        pltpu.sync_copy(stripe_v, acc_sh.at[pl.ds(sid * ROWS, ROWS)])
        pltpu.sync_copy(src2d.at[pl.ds(wid * NCHUNK, NCHUNK)], idx_s)
        pltpu.sync_copy(dst2d.at[pl.ds(wid * NCHUNK, NCHUNK)], idx_d)
        plsc.subcore_barrier()

        ga = pltpu.async_copy(y_hbm.at[idx_s.at[0]], rows_a, sem_a)

        def body(i, carry):
            k0 = 2 * i
            gb = pltpu.async_copy(y_hbm.at[idx_s.at[k0 + 1]], rows_b, sem_b)
            pltpu.make_async_copy(y_hbm.at[idx_s.at[k0]], rows_a, sem_a).wait()
            pltpu.sync_copy(rows_a, acc_sh.at[idx_d.at[k0]], add=True)
            knext = jnp.minimum(k0 + 2, NCHUNK - 1)
            pltpu.async_copy(y_hbm.at[idx_s.at[knext]], rows_a, sem_a)
            gb.wait()
            pltpu.sync_copy(rows_b, acc_sh.at[idx_d.at[k0 + 1]], add=True)
            return carry

        lax.fori_loop(0, NCHUNK // 2, body, 0)
        pltpu.make_async_copy(y_hbm.at[idx_s.at[NCHUNK - 1]], rows_a,
                              sem_a).wait()
        plsc.subcore_barrier()
        pltpu.sync_copy(acc_sh.at[pl.ds(sid * ROWS, ROWS)], stripe_v)
        pltpu.sync_copy(stripe_v, out.at[cid, pl.ds(sid * ROWS, ROWS)])

    return conv_kernel


_deg_kernel = _make_deg_kernel()
_conv16 = _make_conv_kernel(16)
_conv32 = _make_conv_kernel(32)


def _tc1_body(x_ref, w1_ref, degp_ref, y1_ref, dinv_ref):
    deg = degp_ref[:, 0:1] + degp_ref[:, 1:2] + 1.0
    dinv = lax.rsqrt(deg)
    xw = jnp.dot(x_ref[...], w1_ref[...], preferred_element_type=jnp.float32)
    y1_ref[...] = dinv * xw
    dinv_ref[...] = dinv


def _tc2_body(acca_ref, accb_ref, y_ref, dinv_ref, b_ref, w2_ref, y2_ref):
    dinv = dinv_ref[...]
    h = acca_ref[...] + accb_ref[...] + y_ref[...]
    h = jax.nn.relu(dinv * h + b_ref[...])
    y2_ref[...] = dinv * jnp.dot(h, w2_ref[...], preferred_element_type=jnp.float32)


def _tc3_body(acca_ref, accb_ref, y_ref, dinv_ref, b_ref, h_ref):
    h = acca_ref[...] + accb_ref[...] + y_ref[...]
    h_ref[...] = jax.nn.relu(dinv_ref[...] * h + b_ref[...])


def _gi_body(haug_ref, wbig_ref, bcat_ref, gi_ref):
    gi_ref[...] = jnp.dot(haug_ref[...], wbig_ref[...],
                          preferred_element_type=jnp.float32) + bcat_ref[...]


def _scan_body(gi_ref, wc_ref, bhn_ref, hall_ref):
    bhn = bhn_ref[...]

    def step(t, carry):
        h, hbt = carry
        gi_t = gi_ref[pl.ds(t, 1), :]
        s0 = jnp.sum(hbt * wc_ref[:, 0:128], axis=0, keepdims=True)
        s1 = jnp.sum(hbt * wc_ref[:, 128:256], axis=0, keepdims=True)
        s2h = jnp.sum(hbt * wc_ref[:, 256:384], axis=0, keepdims=True) + bhn
        tr = jnp.tanh(gi_t[:, 0:128] + s0)
        tz = jnp.tanh(gi_t[:, 128:256] + s1)
        c = jnp.tanh(gi_t[:, 256:384] + s2h + tr * s2h)
        hn = 0.5 * (c + h) + tz * (0.5 * (h - c))
        hall_ref[pl.ds(t, 1), :] = hn
        hbt_n = jnp.broadcast_to(hn, (128, 128)).T
        return (hn, hbt_n)

    lax.fori_loop(0, N, step, (jnp.zeros((1, 2 * H), jnp.float32),
                               jnp.zeros((128, 128), jnp.float32)),
                  unroll=4)


def _out_body(h_ref, w_ref, b_ref, o_ref):
    o_ref[...] = jnp.dot(h_ref[...], w_ref[...],
                         preferred_element_type=jnp.float32) + b_ref[...]


def _call(body, out_shapes):
    return pl.pallas_call(body, out_shape=out_shapes)


def _gates_cat(Wf, Wr):
    insz = Wf.shape[1]
    Wbig = jnp.zeros((2 * insz, 6 * H), jnp.float32)
    WfT, WrT = Wf.T, Wr.T
    for g in range(3):
        Wbig = Wbig.at[:insz, (2 * g) * H:(2 * g + 1) * H].set(
            WfT[:, g * H:(g + 1) * H])
        Wbig = Wbig.at[insz:, (2 * g + 1) * H:(2 * g + 2) * H].set(
            WrT[:, g * H:(g + 1) * H])
    return Wbig


def _bias_cat(bf, br):
    return jnp.concatenate([bf[0:H], br[0:H], bf[H:2 * H], br[H:2 * H],
                            bf[2 * H:], br[2 * H:]]).reshape(1, 6 * H)


def _scan_prep(wih_f, wih_r, bih_f, bih_r, whh_f, whh_r, bhh_f, bhh_r):
    half = jnp.concatenate([jnp.full((256,), 0.5, jnp.float32),
                            jnp.ones((128,), jnp.float32)]).reshape(1, 384)
    wbig = _gates_cat(wih_f, wih_r) * half
    bi = _bias_cat(bih_f, bih_r)
    bh = _bias_cat(bhh_f, bhh_r)
    gi_bias = jnp.concatenate([0.5 * (bi[:, 0:256] + bh[:, 0:256]),
                               bi[:, 256:384]], axis=1)
    wblk = _gates_cat(whh_f, whh_r)
    wc = 0.5 * wblk
    bhn_half = 0.5 * bh[:, 256:384]
    return wbig, gi_bias, wc, bhn_half


def kernel(x, edge_index, W1, b1, W2, b2,
           gru_wih_l0, gru_whh_l0, gru_bih_l0, gru_bhh_l0,
           gru_wih_l0_r, gru_whh_l0_r, gru_bih_l0_r, gru_bhh_l0_r,
           gru_wih_l1, gru_whh_l1, gru_bih_l1, gru_bhh_l1,
           gru_wih_l1_r, gru_whh_l1_r, gru_bih_l1_r, gru_bhh_l1_r,
           Wlin, blin):
    f32 = jnp.float32
    src = edge_index[0]
    dst = edge_index[1]
    pad = EPAD - E
    src2d = jnp.concatenate([src, jnp.zeros((pad,), jnp.int32)]).reshape(-1, CH)
    dst2d = jnp.concatenate([dst, jnp.full((pad,), N, jnp.int32)]).reshape(-1, CH)

    ones16 = jnp.ones((CH, 16), f32)
    zeros16 = jnp.zeros((NACC, 16), f32)
    zeros32 = jnp.zeros((NACC, 32), f32)

    degp = _deg_kernel(dst2d, ones16, zeros16)
    degp2 = jnp.transpose(degp[:, :N, 0])

    y1, dinv = _call(_tc1_body, [jax.ShapeDtypeStruct((N, 16), f32),
                                 jax.ShapeDtypeStruct((N, 1), f32)])(
        x, W1, degp2)
    acc1 = _conv16(y1, src2d, dst2d, zeros16)
    y2 = _call(_tc2_body, jax.ShapeDtypeStruct((N, 32), f32))(
        acc1[0, :N], acc1[1, :N], y1, dinv, b1.reshape(1, 16), W2)

    acc2 = _conv32(y2, src2d, dst2d, zeros32)
    h = _call(_tc3_body, jax.ShapeDtypeStruct((N, 32), f32))(
        acc2[0, :N], acc2[1, :N], y2, dinv, b2.reshape(1, 32))

    haug = jnp.concatenate([h, h[::-1]], axis=-1)
    wbig0, gib0, wc0, bhn0 = _scan_prep(
        gru_wih_l0, gru_wih_l0_r, gru_bih_l0, gru_bih_l0_r,
        gru_whh_l0, gru_whh_l0_r, gru_bhh_l0, gru_bhh_l0_r)
    gi0 = _call(_gi_body, jax.ShapeDtypeStruct((N, 6 * H), f32))(
        haug, wbig0, gib0)
    hall0 = _call(_scan_body, jax.ShapeDtypeStruct((N, 2 * H), f32))(
        gi0, wc0, bhn0)

    h1 = jnp.concatenate([hall0[:, :H], hall0[::-1, H:]], axis=-1)
    h1aug = jnp.concatenate([h1, h1[::-1]], axis=-1)
    wbig1, gib1, wc1, bhn1 = _scan_prep(
        gru_wih_l1, gru_wih_l1_r, gru_bih_l1, gru_bih_l1_r,
        gru_whh_l1, gru_whh_l1_r, gru_bhh_l1, gru_bhh_l1_r)
    gi1 = _call(_gi_body, jax.ShapeDtypeStruct((N, 6 * H), f32))(
        h1aug, wbig1, gib1)
    hall1 = _call(_scan_body, jax.ShapeDtypeStruct((N, 2 * H), f32))(
        gi1, wc1, bhn1)

    h2cat = jnp.concatenate([hall1[:, :H], hall1[::-1, H:]], axis=-1)
    return _call(_out_body, jax.ShapeDtypeStruct((N, Wlin.shape[1]), f32))(
        h2cat, Wlin, blin.reshape(1, -1))
